# Initial kernel scaffold; baseline (speedup 1.0000x reference)
import jax
import jax.numpy as jnp
from jax.experimental import pallas as pl


def kernel(emb, pos, Wq, Wk, Wv, Wo, W1, W2, tokens):
    raise NotImplementedError("write your pallas kernel here")


# speedup vs baseline: 1.8174x; 1.8174x over previous
import jax
import jax.numpy as jnp
from jax.experimental import pallas as pl
from jax.experimental.pallas import tpu as pltpu

DEPTH = 4
HEADS = 4
DIM = 256
BS = 64
SEQ = 1536
FF = 1024
SINK_ITERS = 8
TEMP = 0.75
NB = SEQ // BS
DH = DIM // HEADS
VOCAB_P = 32
SCALE = DH ** -0.5


def _layer_norm(x, eps=1e-5):
    m = x.mean(-1, keepdims=True)
    v = jnp.var(x, axis=-1, keepdims=True)
    return (x - m) / jnp.sqrt(v + eps)


def _lse(r, axis):
    m = jnp.max(r, axis=axis, keepdims=True)
    return m + jnp.log(jnp.sum(jnp.exp(r - m), axis=axis, keepdims=True))


def _fwd_kernel(tcol_ref, tbkt_ref, emb_ref, pos_ref, wqkv_ref, wo_ref,
                w1_ref, w2_ref, out_ref):
    f32 = jnp.float32
    bf16 = jnp.bfloat16

    tok_c = tcol_ref[0]
    maskc = (tok_c > 0).astype(f32)
    bm = (tbkt_ref[0] > 0).astype(f32)

    oh = (tok_c == jax.lax.broadcasted_iota(jnp.int32, (1, VOCAB_P), 1))
    x = jax.lax.dot_general(oh.astype(f32), emb_ref[...],
                            (((1,), (0,)), ((), ())),
                            preferred_element_type=f32)
    x = x + pos_ref[...]

    denom = jnp.maximum(jnp.sum(bm, axis=-1, keepdims=True), 1.0)

    for i in range(DEPTH):
        ln1 = _layer_norm(x).astype(bf16)
        qkv = jnp.dot(ln1, wqkv_ref[i], preferred_element_type=f32)

        qk_sums = jnp.sum((qkv[:, :2 * DIM] * maskc).reshape(NB, BS, 2 * DIM),
                          axis=1)
        qk_means = qk_sums / denom

        rs = []
        for h in range(HEADS):
            q_m = qk_means[:, h * DH:(h + 1) * DH]
            k_m = qk_means[:, DIM + h * DH:DIM + (h + 1) * DH]
            r_h = jax.lax.dot_general(q_m, k_m, (((1,), (1,)), ((), ())),
                                      preferred_element_type=f32)
            rs.append(r_h * (SCALE / TEMP))
        r = jnp.stack(rs, axis=0)

        for _ in range(SINK_ITERS):
            r = r - _lse(r, -1)
            r = r - _lse(r, -2)
        p_all = jnp.exp(r)

        qkvb = qkv.astype(bf16)
        head_outs = []
        for h in range(HEADS):
            qh = qkvb[:, h * DH:(h + 1) * DH].reshape(NB, BS, DH)
            kh = qkvb[:, DIM + h * DH:DIM + (h + 1) * DH].reshape(NB, BS, DH)
            vh = qkvb[:, 2 * DIM + h * DH:2 * DIM + (h + 1) * DH].reshape(NB, BS, DH)
            p_h = p_all[h]
            p_hb = p_h.astype(bf16)

            sk = jax.lax.dot_general(p_hb, kh, (((1,), (0,)), ((), ())),
                                     preferred_element_type=f32)
            sv = jax.lax.dot_general(p_hb, vh, (((1,), (0,)), ((), ())),
                                     preferred_element_type=f32)
            sm = jnp.dot(p_h, bm, preferred_element_type=f32)

            keys = jnp.concatenate([kh, sk.astype(bf16)], axis=1)
            vals = jnp.concatenate([vh, sv.astype(bf16)], axis=1)
            kmask = jnp.concatenate([bm, jnp.clip(sm, 0.0, 1.0)], axis=-1)
            logm = jnp.log(kmask + 1e-9)

            sc = jax.lax.dot_general(qh, keys, (((2,), (2,)), ((0,), (0,))),
                                     preferred_element_type=f32)
            sc = sc * SCALE + logm[:, None, :]
            mx = jnp.max(sc, axis=-1, keepdims=True)
            e = jnp.exp(sc - mx)
            w = (e / jnp.sum(e, axis=-1, keepdims=True)).astype(bf16)
            o_h = jax.lax.dot_general(w, vals, (((2,), (1,)), ((0,), (0,))),
                                      preferred_element_type=f32)
            head_outs.append(o_h.reshape(SEQ, DH))
        att = jnp.concatenate(head_outs, axis=-1).astype(bf16)
        x = x + jnp.dot(att, wo_ref[i], preferred_element_type=f32)

        ln2 = _layer_norm(x).astype(bf16)
        hmid = jnp.dot(ln2, w1_ref[i], preferred_element_type=f32)
        g = jax.nn.gelu(hmid).astype(bf16)
        x = x + jnp.dot(g, w2_ref[i], preferred_element_type=f32)

    xl = _layer_norm(x)
    cnt = jnp.maximum(jnp.sum(maskc), 1.0)
    out_ref[...] = (jnp.sum(xl * maskc, axis=0, keepdims=True) / cnt)[None]


def kernel(emb, pos, Wq, Wk, Wv, Wo, W1, W2, tokens):
    tokens = tokens.astype(jnp.int32)
    batch = tokens.shape[0]
    tcol = tokens.reshape(batch, SEQ, 1)
    tbkt = tokens.reshape(batch, NB, BS)
    emb_p = jnp.zeros((VOCAB_P, DIM), jnp.float32).at[:emb.shape[0]].set(emb)
    wqkv = jnp.concatenate([Wq, Wk, Wv], axis=-1).astype(jnp.bfloat16)
    wo = Wo.astype(jnp.bfloat16)
    w1 = W1.astype(jnp.bfloat16)
    w2 = W2.astype(jnp.bfloat16)
    pos_f = pos.astype(jnp.float32)

    return pl.pallas_call(
        _fwd_kernel,
        grid=(batch,),
        in_specs=[
            pl.BlockSpec((1, SEQ, 1), lambda b: (b, 0, 0)),
            pl.BlockSpec((1, NB, BS), lambda b: (b, 0, 0)),
            pl.BlockSpec((VOCAB_P, DIM), lambda b: (0, 0)),
            pl.BlockSpec((SEQ, DIM), lambda b: (0, 0)),
            pl.BlockSpec((DEPTH, DIM, 3 * DIM), lambda b: (0, 0, 0)),
            pl.BlockSpec((DEPTH, DIM, DIM), lambda b: (0, 0, 0)),
            pl.BlockSpec((DEPTH, DIM, FF), lambda b: (0, 0, 0)),
            pl.BlockSpec((DEPTH, FF, DIM), lambda b: (0, 0, 0)),
        ],
        out_specs=pl.BlockSpec((1, 1, DIM), lambda b: (b, 0, 0)),
        out_shape=jax.ShapeDtypeStruct((batch, 1, DIM), jnp.float32),
        compiler_params=pltpu.CompilerParams(
            dimension_semantics=("arbitrary",),
        ),
    )(tcol, tbkt, emb_p, pos_f, wqkv, wo, w1, w2).reshape(batch, DIM)


# speedup vs baseline: 2.4630x; 1.3553x over previous
import jax
import jax.numpy as jnp
from jax.experimental import pallas as pl
from jax.experimental.pallas import tpu as pltpu

DEPTH = 4
HEADS = 4
DIM = 256
BS = 64
SEQ = 1536
FF = 1024
SINK_ITERS = 8
TEMP = 0.75
NB = SEQ // BS
DH = DIM // HEADS
VOCAB_P = 32
SCALE = DH ** -0.5


def _layer_norm(x, eps=1e-5):
    m = x.mean(-1, keepdims=True)
    v = jnp.var(x, axis=-1, keepdims=True)
    return (x - m) / jnp.sqrt(v + eps)


def _lse(r, axis):
    m = jnp.max(r, axis=axis, keepdims=True)
    return m + jnp.log(jnp.sum(jnp.exp(r - m), axis=axis, keepdims=True))


def _fwd_kernel(tcol_ref, tbkt_ref, emb_ref, pos_ref, wqkv_ref, wo_ref,
                w1_ref, w2_ref, out_ref):
    f32 = jnp.float32
    bf16 = jnp.bfloat16

    tok_c = tcol_ref[0]
    maskc = (tok_c > 0).astype(f32)
    bm = (tbkt_ref[0] > 0).astype(f32)

    oh = (tok_c == jax.lax.broadcasted_iota(jnp.int32, (1, VOCAB_P), 1))
    x = jax.lax.dot_general(oh.astype(f32), emb_ref[...],
                            (((1,), (0,)), ((), ())),
                            preferred_element_type=f32)
    x = x + pos_ref[...]

    denom = jnp.maximum(jnp.sum(bm, axis=-1, keepdims=True), 1.0)

    for i in range(DEPTH):
        ln1 = _layer_norm(x).astype(bf16)
        qkv = jnp.dot(ln1, wqkv_ref[i], preferred_element_type=f32)

        qk_sums = jnp.sum((qkv[:, :2 * DIM] * maskc).reshape(NB, BS, 2 * DIM),
                          axis=1)
        qk_means = qk_sums / denom

        rs = []
        for h in range(HEADS):
            q_m = qk_means[:, h * DH:(h + 1) * DH]
            k_m = qk_means[:, DIM + h * DH:DIM + (h + 1) * DH]
            r_h = jax.lax.dot_general(q_m, k_m, (((1,), (1,)), ((), ())),
                                      preferred_element_type=f32)
            rs.append(r_h * (SCALE / TEMP))
        r = jnp.stack(rs, axis=0)

        p_all = jnp.exp(r - jnp.max(r, axis=-1, keepdims=True))
        for _ in range(SINK_ITERS):
            p_all = p_all / jnp.sum(p_all, axis=-1, keepdims=True)
            p_all = p_all / jnp.sum(p_all, axis=-2, keepdims=True)

        qkvb = qkv.astype(bf16)
        head_outs = []
        for h in range(HEADS):
            qh = qkvb[:, h * DH:(h + 1) * DH].reshape(NB, BS, DH)
            kh = qkvb[:, DIM + h * DH:DIM + (h + 1) * DH].reshape(NB, BS, DH)
            vh = qkvb[:, 2 * DIM + h * DH:2 * DIM + (h + 1) * DH].reshape(NB, BS, DH)
            p_h = p_all[h]
            p_hb = p_h.astype(bf16)

            khv = jnp.concatenate([kh, vh], axis=-1)
            skv = jax.lax.dot_general(p_hb, khv, (((1,), (0,)), ((), ())),
                                      preferred_element_type=f32).astype(bf16)
            sm = jnp.dot(p_h, bm, preferred_element_type=f32)

            keys = jnp.concatenate([kh, skv[..., :DH]], axis=1)
            vals = jnp.concatenate([vh, skv[..., DH:]], axis=1)
            kmask = jnp.concatenate([bm, jnp.clip(sm, 0.0, 1.0)], axis=-1)
            logm = jnp.log(kmask + 1e-9)

            sc = jax.lax.dot_general(qh, keys, (((2,), (2,)), ((0,), (0,))),
                                     preferred_element_type=f32)
            sc = sc * SCALE + logm[:, None, :]
            mx = jnp.max(sc, axis=-1, keepdims=True)
            e = jnp.exp(sc - mx)
            w = (e / jnp.sum(e, axis=-1, keepdims=True)).astype(bf16)
            o_h = jax.lax.dot_general(w, vals, (((2,), (1,)), ((0,), (0,))),
                                      preferred_element_type=f32)
            head_outs.append(o_h.reshape(SEQ, DH))
        att = jnp.concatenate(head_outs, axis=-1).astype(bf16)
        x = x + jnp.dot(att, wo_ref[i], preferred_element_type=f32)

        ln2 = _layer_norm(x).astype(bf16)
        hmid = jnp.dot(ln2, w1_ref[i], preferred_element_type=f32)
        g = jax.nn.gelu(hmid).astype(bf16)
        x = x + jnp.dot(g, w2_ref[i], preferred_element_type=f32)

    xl = _layer_norm(x)
    cnt = jnp.maximum(jnp.sum(maskc), 1.0)
    out_ref[...] = (jnp.sum(xl * maskc, axis=0, keepdims=True) / cnt)[None]


def kernel(emb, pos, Wq, Wk, Wv, Wo, W1, W2, tokens):
    tokens = tokens.astype(jnp.int32)
    batch = tokens.shape[0]
    tcol = tokens.reshape(batch, SEQ, 1)
    tbkt = tokens.reshape(batch, NB, BS)
    emb_p = jnp.zeros((VOCAB_P, DIM), jnp.float32).at[:emb.shape[0]].set(emb)
    wqkv = jnp.concatenate([Wq, Wk, Wv], axis=-1).astype(jnp.bfloat16)
    wo = Wo.astype(jnp.bfloat16)
    w1 = W1.astype(jnp.bfloat16)
    w2 = W2.astype(jnp.bfloat16)
    pos_f = pos.astype(jnp.float32)

    return pl.pallas_call(
        _fwd_kernel,
        grid=(batch,),
        in_specs=[
            pl.BlockSpec((1, SEQ, 1), lambda b: (b, 0, 0)),
            pl.BlockSpec((1, NB, BS), lambda b: (b, 0, 0)),
            pl.BlockSpec((VOCAB_P, DIM), lambda b: (0, 0)),
            pl.BlockSpec((SEQ, DIM), lambda b: (0, 0)),
            pl.BlockSpec((DEPTH, DIM, 3 * DIM), lambda b: (0, 0, 0)),
            pl.BlockSpec((DEPTH, DIM, DIM), lambda b: (0, 0, 0)),
            pl.BlockSpec((DEPTH, DIM, FF), lambda b: (0, 0, 0)),
            pl.BlockSpec((DEPTH, FF, DIM), lambda b: (0, 0, 0)),
        ],
        out_specs=pl.BlockSpec((1, 1, DIM), lambda b: (b, 0, 0)),
        out_shape=jax.ShapeDtypeStruct((batch, 1, DIM), jnp.float32),
        compiler_params=pltpu.CompilerParams(
            dimension_semantics=("arbitrary",),
        ),
    )(tcol, tbkt, emb_p, pos_f, wqkv, wo, w1, w2).reshape(batch, DIM)


# speedup vs baseline: 2.9521x; 1.1986x over previous
import jax
import jax.numpy as jnp
from jax.experimental import pallas as pl
from jax.experimental.pallas import tpu as pltpu

DEPTH = 4
HEADS = 4
DIM = 256
BS = 64
SEQ = 1536
FF = 1024
SINK_ITERS = 8
TEMP = 0.75
NB = SEQ // BS
DH = DIM // HEADS
VOCAB_P = 32
SCALE = DH ** -0.5


def _layer_norm(x, eps=1e-5):
    m = x.mean(-1, keepdims=True)
    v = jnp.var(x, axis=-1, keepdims=True)
    return (x - m) / jnp.sqrt(v + eps)


def _lse(r, axis):
    m = jnp.max(r, axis=axis, keepdims=True)
    return m + jnp.log(jnp.sum(jnp.exp(r - m), axis=axis, keepdims=True))


def _fwd_kernel(tcol_ref, tbkt_ref, emb_ref, pos_ref, wqkv_ref, wo_ref,
                w1_ref, w2_ref, out_ref):
    f32 = jnp.float32
    bf16 = jnp.bfloat16

    tok_c = tcol_ref[0]
    maskc = (tok_c > 0).astype(f32)
    bm = (tbkt_ref[0] > 0).astype(f32)

    oh = (tok_c == jax.lax.broadcasted_iota(jnp.int32, (1, VOCAB_P), 1))
    x = jax.lax.dot_general(oh.astype(f32), emb_ref[...],
                            (((1,), (0,)), ((), ())),
                            preferred_element_type=f32)
    x = x + pos_ref[...]

    denom = jnp.maximum(jnp.sum(bm, axis=-1, keepdims=True), 1.0)

    for i in range(DEPTH):
        ln1 = _layer_norm(x).astype(bf16)
        qkv = jnp.dot(ln1, wqkv_ref[i], preferred_element_type=f32)

        sums = jnp.sum((qkv * maskc).reshape(NB, BS, 3 * DIM), axis=1)
        means = sums / denom

        rs = []
        for h in range(HEADS):
            q_m = means[:, h * DH:(h + 1) * DH]
            k_m = means[:, DIM + h * 2 * DH:DIM + h * 2 * DH + DH]
            r_h = jax.lax.dot_general(q_m, k_m, (((1,), (1,)), ((), ())),
                                      preferred_element_type=f32)
            rs.append(r_h * (SCALE / TEMP))
        r = jnp.stack(rs, axis=0)

        p_all = jnp.exp(r - jnp.max(r, axis=-1, keepdims=True))
        for _ in range(SINK_ITERS):
            p_all = p_all / jnp.sum(p_all, axis=-1, keepdims=True)
            p_all = p_all / jnp.sum(p_all, axis=-2, keepdims=True)

        qkvb = qkv.astype(bf16)
        q_all = jnp.concatenate(
            [qkvb[:, h * DH:(h + 1) * DH].reshape(NB, BS, DH)
             for h in range(HEADS)], axis=0)
        khv_all = jnp.concatenate(
            [qkvb[:, DIM + h * 2 * DH:DIM + (h + 1) * 2 * DH].reshape(NB, BS, 2 * DH)
             for h in range(HEADS)], axis=0)

        p_b = p_all.astype(bf16)
        zb = jnp.zeros((NB, NB), bf16)
        p_blk = jnp.concatenate(
            [jnp.concatenate([p_b[h] if j == h else zb for j in range(HEADS)],
                             axis=-1) for h in range(HEADS)], axis=0)
        skv_all = jax.lax.dot_general(p_blk, khv_all, (((1,), (0,)), ((), ())),
                                      preferred_element_type=f32).astype(bf16)

        p_stack = jnp.concatenate([p_all[h] for h in range(HEADS)], axis=0)
        sm_all = jnp.dot(p_stack, bm, preferred_element_type=f32)

        keys = jnp.concatenate([khv_all[..., :DH], skv_all[..., :DH]], axis=1)
        vals = jnp.concatenate([khv_all[..., DH:], skv_all[..., DH:]], axis=1)
        bm4 = jnp.concatenate([bm] * HEADS, axis=0)
        kmask = jnp.concatenate([bm4, jnp.clip(sm_all, 0.0, 1.0)], axis=-1)
        logm = jnp.log(kmask + 1e-9)

        sc = jax.lax.dot_general(q_all, keys, (((2,), (2,)), ((0,), (0,))),
                                 preferred_element_type=f32)
        sc = sc * SCALE + logm[:, None, :]
        e = jnp.exp(sc)
        s = jnp.sum(e, axis=-1, keepdims=True)
        o_un = jax.lax.dot_general(e.astype(bf16), vals, (((2,), (1,)), ((0,), (0,))),
                                   preferred_element_type=f32)
        o_all = o_un / s
        att = jnp.concatenate(
            [o_all[h * NB:(h + 1) * NB].reshape(SEQ, DH) for h in range(HEADS)],
            axis=-1).astype(bf16)
        x = x + jnp.dot(att, wo_ref[i], preferred_element_type=f32)

        ln2 = _layer_norm(x).astype(bf16)
        hmid = jnp.dot(ln2, w1_ref[i], preferred_element_type=f32)
        g = jax.nn.gelu(hmid).astype(bf16)
        x = x + jnp.dot(g, w2_ref[i], preferred_element_type=f32)

    xl = _layer_norm(x)
    cnt = jnp.maximum(jnp.sum(maskc), 1.0)
    out_ref[...] = (jnp.sum(xl * maskc, axis=0, keepdims=True) / cnt)[None]


def kernel(emb, pos, Wq, Wk, Wv, Wo, W1, W2, tokens):
    tokens = tokens.astype(jnp.int32)
    batch = tokens.shape[0]
    tcol = tokens.reshape(batch, SEQ, 1)
    tbkt = tokens.reshape(batch, NB, BS)
    emb_p = jnp.zeros((VOCAB_P, DIM), jnp.float32).at[:emb.shape[0]].set(emb)
    kv_cols = [jnp.concatenate([Wk[:, :, h * DH:(h + 1) * DH],
                                Wv[:, :, h * DH:(h + 1) * DH]], axis=-1)
               for h in range(HEADS)]
    wqkv = jnp.concatenate([Wq] + kv_cols, axis=-1).astype(jnp.bfloat16)
    wo = Wo.astype(jnp.bfloat16)
    w1 = W1.astype(jnp.bfloat16)
    w2 = W2.astype(jnp.bfloat16)
    pos_f = pos.astype(jnp.float32)

    return pl.pallas_call(
        _fwd_kernel,
        grid=(batch,),
        in_specs=[
            pl.BlockSpec((1, SEQ, 1), lambda b: (b, 0, 0)),
            pl.BlockSpec((1, NB, BS), lambda b: (b, 0, 0)),
            pl.BlockSpec((VOCAB_P, DIM), lambda b: (0, 0)),
            pl.BlockSpec((SEQ, DIM), lambda b: (0, 0)),
            pl.BlockSpec((DEPTH, DIM, 3 * DIM), lambda b: (0, 0, 0)),
            pl.BlockSpec((DEPTH, DIM, DIM), lambda b: (0, 0, 0)),
            pl.BlockSpec((DEPTH, DIM, FF), lambda b: (0, 0, 0)),
            pl.BlockSpec((DEPTH, FF, DIM), lambda b: (0, 0, 0)),
        ],
        out_specs=pl.BlockSpec((1, 1, DIM), lambda b: (b, 0, 0)),
        out_shape=jax.ShapeDtypeStruct((batch, 1, DIM), jnp.float32),
        compiler_params=pltpu.CompilerParams(
            dimension_semantics=("arbitrary",),
        ),
    )(tcol, tbkt, emb_p, pos_f, wqkv, wo, w1, w2).reshape(batch, DIM)


# speedup vs baseline: 2.9792x; 1.0092x over previous
import jax
import jax.numpy as jnp
from jax.experimental import pallas as pl
from jax.experimental.pallas import tpu as pltpu

DEPTH = 4
HEADS = 4
DIM = 256
BS = 64
SEQ = 1536
FF = 1024
SINK_ITERS = 8
TEMP = 0.75
NB = SEQ // BS
DH = DIM // HEADS
VOCAB_P = 32
SCALE = DH ** -0.5


def _layer_norm(x, eps=1e-5):
    m = x.mean(-1, keepdims=True)
    v = jnp.var(x, axis=-1, keepdims=True)
    return (x - m) / jnp.sqrt(v + eps)


def _lse(r, axis):
    m = jnp.max(r, axis=axis, keepdims=True)
    return m + jnp.log(jnp.sum(jnp.exp(r - m), axis=axis, keepdims=True))


def _fwd_kernel(tcol_ref, trow_ref, tbkt_ref, emb_ref, pos_ref, wqkv_ref,
                wo_ref, w1_ref, w2_ref, out_ref):
    f32 = jnp.float32
    bf16 = jnp.bfloat16

    tok_c = tcol_ref[0]
    maskc = (tok_c > 0).astype(f32)
    bm = (tbkt_ref[0] > 0).astype(f32)

    rowid = jax.lax.broadcasted_iota(jnp.int32, (NB, 1), 0)
    t_bkt = jax.lax.broadcasted_iota(jnp.int32, (1, SEQ), 1) // BS
    maskr = (trow_ref[0] > 0)
    bsel = ((rowid == t_bkt) & maskr).astype(bf16)

    oh = (tok_c == jax.lax.broadcasted_iota(jnp.int32, (1, VOCAB_P), 1))
    x = jax.lax.dot_general(oh.astype(f32), emb_ref[...],
                            (((1,), (0,)), ((), ())),
                            preferred_element_type=f32)
    x = x + pos_ref[...]

    denom = jnp.maximum(jnp.sum(bm, axis=-1, keepdims=True), 1.0)

    for i in range(DEPTH):
        ln1 = _layer_norm(x).astype(bf16)
        qkv = jnp.dot(ln1, wqkv_ref[i], preferred_element_type=f32)
        qkvb = qkv.astype(bf16)

        sums = jnp.dot(bsel, qkvb, preferred_element_type=f32)
        means = sums / denom

        rs = []
        for h in range(HEADS):
            q_m = means[:, h * DH:(h + 1) * DH]
            k_m = means[:, DIM + h * 2 * DH:DIM + h * 2 * DH + DH]
            r_h = jax.lax.dot_general(q_m, k_m, (((1,), (1,)), ((), ())),
                                      preferred_element_type=f32)
            rs.append(r_h * (SCALE / TEMP))
        r = jnp.stack(rs, axis=0)

        p_all = jnp.exp(r - jnp.max(r, axis=-1, keepdims=True))
        for _ in range(SINK_ITERS):
            p_all = p_all / jnp.sum(p_all, axis=-1, keepdims=True)
            p_all = p_all / jnp.sum(p_all, axis=-2, keepdims=True)

        q_all = jnp.concatenate(
            [qkvb[:, h * DH:(h + 1) * DH].reshape(NB, BS, DH)
             for h in range(HEADS)], axis=0)
        khv_all = jnp.concatenate(
            [qkvb[:, DIM + h * 2 * DH:DIM + (h + 1) * 2 * DH].reshape(NB, BS, 2 * DH)
             for h in range(HEADS)], axis=0)

        p_b = p_all.astype(bf16)
        zb = jnp.zeros((NB, NB), bf16)
        p_blk = jnp.concatenate(
            [jnp.concatenate([p_b[h] if j == h else zb for j in range(HEADS)],
                             axis=-1) for h in range(HEADS)], axis=0)
        skv_all = jax.lax.dot_general(p_blk, khv_all, (((1,), (0,)), ((), ())),
                                      preferred_element_type=f32).astype(bf16)

        p_stack = jnp.concatenate([p_all[h] for h in range(HEADS)], axis=0)
        sm_all = jnp.dot(p_stack, bm, preferred_element_type=f32)

        keys = jnp.concatenate([khv_all[..., :DH], skv_all[..., :DH]], axis=1)
        vals = jnp.concatenate([khv_all[..., DH:], skv_all[..., DH:]], axis=1)
        bm4 = jnp.concatenate([bm] * HEADS, axis=0)
        kmask = jnp.concatenate([bm4, jnp.clip(sm_all, 0.0, 1.0)], axis=-1)
        logm = jnp.log(kmask + 1e-9)

        sc = jax.lax.dot_general(q_all, keys, (((2,), (2,)), ((0,), (0,))),
                                 preferred_element_type=f32)
        sc = sc * SCALE + logm[:, None, :]
        e = jnp.exp(sc)
        s = jnp.sum(e, axis=-1, keepdims=True)
        o_un = jax.lax.dot_general(e.astype(bf16), vals, (((2,), (1,)), ((0,), (0,))),
                                   preferred_element_type=f32)
        o_all = o_un / s
        att = jnp.concatenate(
            [o_all[h * NB:(h + 1) * NB].reshape(SEQ, DH) for h in range(HEADS)],
            axis=-1).astype(bf16)
        x = x + jnp.dot(att, wo_ref[i], preferred_element_type=f32)

        ln2 = _layer_norm(x).astype(bf16)
        hmid = jnp.dot(ln2, w1_ref[i], preferred_element_type=f32).astype(bf16)
        g = jax.nn.gelu(hmid)
        x = x + jnp.dot(g, w2_ref[i], preferred_element_type=f32)

    xl = _layer_norm(x)
    cnt = jnp.maximum(jnp.sum(maskc), 1.0)
    out_ref[...] = (jnp.sum(xl * maskc, axis=0, keepdims=True) / cnt)[None]


def kernel(emb, pos, Wq, Wk, Wv, Wo, W1, W2, tokens):
    tokens = tokens.astype(jnp.int32)
    batch = tokens.shape[0]
    tcol = tokens.reshape(batch, SEQ, 1)
    trow = tokens.reshape(batch, 1, SEQ)
    tbkt = tokens.reshape(batch, NB, BS)
    emb_p = jnp.zeros((VOCAB_P, DIM), jnp.float32).at[:emb.shape[0]].set(emb)
    kv_cols = [jnp.concatenate([Wk[:, :, h * DH:(h + 1) * DH],
                                Wv[:, :, h * DH:(h + 1) * DH]], axis=-1)
               for h in range(HEADS)]
    wqkv = jnp.concatenate([Wq] + kv_cols, axis=-1).astype(jnp.bfloat16)
    wo = Wo.astype(jnp.bfloat16)
    w1 = W1.astype(jnp.bfloat16)
    w2 = W2.astype(jnp.bfloat16)
    pos_f = pos.astype(jnp.float32)

    return pl.pallas_call(
        _fwd_kernel,
        grid=(batch,),
        in_specs=[
            pl.BlockSpec((1, SEQ, 1), lambda b: (b, 0, 0)),
            pl.BlockSpec((1, 1, SEQ), lambda b: (b, 0, 0)),
            pl.BlockSpec((1, NB, BS), lambda b: (b, 0, 0)),
            pl.BlockSpec((VOCAB_P, DIM), lambda b: (0, 0)),
            pl.BlockSpec((SEQ, DIM), lambda b: (0, 0)),
            pl.BlockSpec((DEPTH, DIM, 3 * DIM), lambda b: (0, 0, 0)),
            pl.BlockSpec((DEPTH, DIM, DIM), lambda b: (0, 0, 0)),
            pl.BlockSpec((DEPTH, DIM, FF), lambda b: (0, 0, 0)),
            pl.BlockSpec((DEPTH, FF, DIM), lambda b: (0, 0, 0)),
        ],
        out_specs=pl.BlockSpec((1, 1, DIM), lambda b: (b, 0, 0)),
        out_shape=jax.ShapeDtypeStruct((batch, 1, DIM), jnp.float32),
        compiler_params=pltpu.CompilerParams(
            dimension_semantics=("arbitrary",),
        ),
    )(tcol, trow, tbkt, emb_p, pos_f, wqkv, wo, w1, w2).reshape(batch, DIM)


# speedup vs baseline: 3.3068x; 1.1100x over previous
import jax
import jax.numpy as jnp
from jax.experimental import pallas as pl
from jax.experimental.pallas import tpu as pltpu

DEPTH = 4
HEADS = 4
DIM = 256
BS = 64
SEQ = 1536
FF = 1024
SINK_ITERS = 8
TEMP = 0.75
NB = SEQ // BS
DH = DIM // HEADS
VOCAB_P = 32
SCALE = DH ** -0.5
MB = 2
SEQ2 = MB * SEQ
NB2 = MB * NB
G = MB * HEADS


def _layer_norm(x, eps=1e-5):
    m = x.mean(-1, keepdims=True)
    v = jnp.var(x, axis=-1, keepdims=True)
    return (x - m) / jnp.sqrt(v + eps)


def _fwd_kernel(tcol_ref, trow_ref, tbkt_ref, emb_ref, pos_ref, wqkv_ref,
                wo_ref, w1_ref, w2_ref, out_ref):
    f32 = jnp.float32
    bf16 = jnp.bfloat16

    tok_c = tcol_ref[0]
    maskc = (tok_c > 0).astype(f32)
    bm = (tbkt_ref[0] > 0).astype(f32)
    maskr = trow_ref[0] > 0

    rowid = jax.lax.broadcasted_iota(jnp.int32, (NB2, 1), 0)
    t_bkt = jax.lax.broadcasted_iota(jnp.int32, (1, SEQ2), 1) // BS
    bsel = ((rowid == t_bkt) & maskr).astype(bf16)

    erow = jax.lax.broadcasted_iota(jnp.int32, (MB, 1), 0)
    t_el = jax.lax.broadcasted_iota(jnp.int32, (1, SEQ2), 1) // SEQ
    psel = ((erow == t_el) & maskr).astype(bf16)
    cnt = jnp.maximum(jnp.sum(psel.astype(f32), axis=-1, keepdims=True), 1.0)

    oh = (tok_c == jax.lax.broadcasted_iota(jnp.int32, (1, VOCAB_P), 1))
    x = jax.lax.dot_general(oh.astype(f32), emb_ref[...],
                            (((1,), (0,)), ((), ())),
                            preferred_element_type=f32)
    pos2 = jnp.concatenate([pos_ref[...]] * MB, axis=0)
    x = x + pos2

    denom = jnp.maximum(jnp.sum(bm, axis=-1, keepdims=True), 1.0)

    for i in range(DEPTH):
        ln1 = _layer_norm(x).astype(bf16)
        qkv = jnp.dot(ln1, wqkv_ref[i], preferred_element_type=f32)
        qkvb = qkv.astype(bf16)

        sums = jnp.dot(bsel, qkvb, preferred_element_type=f32)
        means = sums / denom

        rs = []
        for e in range(MB):
            for h in range(HEADS):
                q_m = means[e * NB:(e + 1) * NB, h * DH:(h + 1) * DH]
                k_m = means[e * NB:(e + 1) * NB,
                            DIM + h * 2 * DH:DIM + h * 2 * DH + DH]
                r_eh = jax.lax.dot_general(q_m, k_m, (((1,), (1,)), ((), ())),
                                           preferred_element_type=f32)
                rs.append(r_eh * (SCALE / TEMP))
        r = jnp.stack(rs, axis=0)

        p_all = jnp.exp(r - jnp.max(r, axis=-1, keepdims=True))
        for _ in range(SINK_ITERS):
            p_all = p_all / jnp.sum(p_all, axis=-1, keepdims=True)
            p_all = p_all / jnp.sum(p_all, axis=-2, keepdims=True)

        q_all = jnp.concatenate(
            [qkvb[e * SEQ:(e + 1) * SEQ, h * DH:(h + 1) * DH].reshape(NB, BS, DH)
             for e in range(MB) for h in range(HEADS)], axis=0)
        khv_all = jnp.concatenate(
            [qkvb[e * SEQ:(e + 1) * SEQ,
                  DIM + h * 2 * DH:DIM + (h + 1) * 2 * DH].reshape(NB, BS, 2 * DH)
             for e in range(MB) for h in range(HEADS)], axis=0)

        p_b = p_all.astype(bf16)
        zb = jnp.zeros((NB, NB), bf16)
        p_blk = jnp.concatenate(
            [jnp.concatenate([p_b[g] if j == g else zb for j in range(G)],
                             axis=-1) for g in range(G)], axis=0)
        skv_all = jax.lax.dot_general(p_blk, khv_all, (((1,), (0,)), ((), ())),
                                      preferred_element_type=f32).astype(bf16)

        znb = jnp.zeros((NB, NB), f32)
        p_stack = jnp.concatenate(
            [jnp.concatenate([p_all[e * HEADS + h] if j == e else znb
                              for j in range(MB)], axis=-1)
             for e in range(MB) for h in range(HEADS)], axis=0)
        sm_all = jnp.dot(p_stack, bm, preferred_element_type=f32)

        keys = jnp.concatenate([khv_all[..., :DH], skv_all[..., :DH]], axis=1)
        vals = jnp.concatenate([khv_all[..., DH:], skv_all[..., DH:]], axis=1)
        bmg = jnp.concatenate([bm[e * NB:(e + 1) * NB] for e in range(MB)
                               for h in range(HEADS)], axis=0)
        kmask = jnp.concatenate([bmg, jnp.clip(sm_all, 0.0, 1.0)], axis=-1)
        logm = jnp.log(kmask + 1e-9)

        sc = jax.lax.dot_general(q_all, keys, (((2,), (2,)), ((0,), (0,))),
                                 preferred_element_type=f32)
        sc = sc * SCALE + logm[:, None, :]
        e_w = jnp.exp(sc)
        s = jnp.sum(e_w, axis=-1, keepdims=True)
        o_un = jax.lax.dot_general(e_w.astype(bf16), vals,
                                   (((2,), (1,)), ((0,), (0,))),
                                   preferred_element_type=f32)
        o_all = o_un / s
        att = jnp.concatenate(
            [jnp.concatenate(
                [o_all[(e * HEADS + h) * NB:(e * HEADS + h + 1) * NB
                       ].reshape(SEQ, DH) for e in range(MB)], axis=0)
             for h in range(HEADS)], axis=-1).astype(bf16)
        x = x + jnp.dot(att, wo_ref[i], preferred_element_type=f32)

        ln2 = _layer_norm(x).astype(bf16)
        hmid = jnp.dot(ln2, w1_ref[i], preferred_element_type=f32).astype(bf16)
        g = jax.nn.gelu(hmid)
        x = x + jnp.dot(g, w2_ref[i], preferred_element_type=f32)

    xl = _layer_norm(x).astype(bf16)
    pooled = jnp.dot(psel, xl, preferred_element_type=f32) / cnt
    out_ref[...] = pooled[:, None, :]


def kernel(emb, pos, Wq, Wk, Wv, Wo, W1, W2, tokens):
    tokens = tokens.astype(jnp.int32)
    batch = tokens.shape[0]
    assert batch % MB == 0
    nsteps = batch // MB
    tcol = tokens.reshape(nsteps, SEQ2, 1)
    trow = tokens.reshape(nsteps, 1, SEQ2)
    tbkt = tokens.reshape(nsteps, NB2, BS)
    emb_p = jnp.zeros((VOCAB_P, DIM), jnp.float32).at[:emb.shape[0]].set(emb)
    kv_cols = [jnp.concatenate([Wk[:, :, h * DH:(h + 1) * DH],
                                Wv[:, :, h * DH:(h + 1) * DH]], axis=-1)
               for h in range(HEADS)]
    wqkv = jnp.concatenate([Wq] + kv_cols, axis=-1).astype(jnp.bfloat16)
    wo = Wo.astype(jnp.bfloat16)
    w1 = W1.astype(jnp.bfloat16)
    w2 = W2.astype(jnp.bfloat16)
    pos_f = pos.astype(jnp.float32)

    return pl.pallas_call(
        _fwd_kernel,
        grid=(nsteps,),
        in_specs=[
            pl.BlockSpec((1, SEQ2, 1), lambda b: (b, 0, 0)),
            pl.BlockSpec((1, 1, SEQ2), lambda b: (b, 0, 0)),
            pl.BlockSpec((1, NB2, BS), lambda b: (b, 0, 0)),
            pl.BlockSpec((VOCAB_P, DIM), lambda b: (0, 0)),
            pl.BlockSpec((SEQ, DIM), lambda b: (0, 0)),
            pl.BlockSpec((DEPTH, DIM, 3 * DIM), lambda b: (0, 0, 0)),
            pl.BlockSpec((DEPTH, DIM, DIM), lambda b: (0, 0, 0)),
            pl.BlockSpec((DEPTH, DIM, FF), lambda b: (0, 0, 0)),
            pl.BlockSpec((DEPTH, FF, DIM), lambda b: (0, 0, 0)),
        ],
        out_specs=pl.BlockSpec((MB, 1, DIM), lambda b: (b, 0, 0)),
        out_shape=jax.ShapeDtypeStruct((batch, 1, DIM), jnp.float32),
        compiler_params=pltpu.CompilerParams(
            dimension_semantics=("arbitrary",),
        ),
    )(tcol, trow, tbkt, emb_p, pos_f, wqkv, wo, w1, w2).reshape(batch, DIM)
